# baseline (speedup 1.0000x reference)
import functools

import jax
import jax.numpy as jnp
from jax import lax
from jax.experimental import pallas as pl
from jax.experimental.pallas import tpu as pltpu
from jax.experimental.pallas import tpu_sc as plsc

_M = 256
_J = 4096
_K = 32
_NNZ = 13107
_NNZ_PAD = 13120
_EXT = _NNZ_PAD + 16
_NTILES = 32
_JT = _J // _NTILES
_CH = _JT * _K
_S1 = 12

_JB = 512

_WCH = 2048


def _sc_scatter_body(fav_hbm, swv_hbm, off_hbm,
                     as_hbm, vs_hbm, cs_hbm, fl_hbm,
                     f_v, av_v, sw_v, v_v, off_v, rs_v, as_v, vs_v, cs_v,
                     sem, sem2):
    nc = 2
    wid = lax.axis_index("s") * nc + lax.axis_index("c")
    base = wid * _CH
    c_off = pltpu.async_copy(off_hbm, off_v, sem2)
    c_sw = pltpu.async_copy(swv_hbm.at[pl.ds(base, _CH)], sw_v, sem)
    c_v = pltpu.async_copy(swv_hbm.at[pl.ds(_K * _J + base, _CH)], v_v, sem)

    zf = jnp.zeros((16,), jnp.float32)
    zi = jnp.zeros((16,), jnp.int32)

    def zero_body(i, c):
        r = i >> 3
        col = (i & 7) * 16
        as_v[r, pl.ds(col, 16)] = zf
        vs_v[r, pl.ds(col, 16)] = zf
        cs_v[r, pl.ds(col, 16)] = zi
        return c

    lax.fori_loop(0, _K * _JT // 16, zero_body, 0)
    c_off.wait()

    offs = off_v[pl.ds(wid, 16)]
    lo = offs[0]
    hi = offs[1]
    lo_fl = (lo // 16) * 16
    nch = (hi - lo_fl + _WCH - 1) // _WCH
    basev = jnp.full((16,), base, jnp.int32)
    chv = jnp.full((16,), _CH, jnp.int32)
    lanes = lax.iota(jnp.int32, 16)

    def chunkA(c, carry):
        oc = jnp.minimum(lo_fl + c * _WCH, _EXT - (_WCH + 16))
        pltpu.sync_copy(fav_hbm.at[pl.ds(oc, _WCH + 16)], f_v)
        s_hi = jnp.minimum((hi - oc + 15) // 16, _WCH // 16)

        def body(s, cc):
            cur = f_v[pl.ds(16 + s * 16, 16)]
            prv = f_v[pl.ds(15 + s * 16, 16)]
            loc = cur - basev
            m = (loc >= 0) & (loc < chv)
            jj = jnp.right_shift(jnp.where(m, loc, 0), 5)
            first = m & (jnp.right_shift(cur, 5) != jnp.right_shift(prv, 5))
            ivec = jnp.full((16,), oc + s * 16, jnp.int32) + lanes
            plsc.store_scatter(rs_v, [jj], ivec, mask=first)
            return cc

        lax.fori_loop(0, s_hi, body, 0)
        return carry

    lax.fori_loop(0, nch, chunkA, 0)
    c_sw.wait()
    c_v.wait()

    def chunkB(c, ovf):
        oc = jnp.minimum(lo_fl + c * _WCH, _EXT - (_WCH + 16))

        @pl.when(nch > 1)
        def _refetch():
            pltpu.sync_copy(fav_hbm.at[pl.ds(oc, _WCH + 16)], f_v)

        pltpu.sync_copy(fav_hbm.at[pl.ds(_EXT + oc, _WCH + 16)], av_v)
        s_hi = jnp.minimum((hi - oc + 15) // 16, _WCH // 16)

        def body(s, ovf_in):
            cur = f_v[pl.ds(16 + s * 16, 16)]
            a = plsc.bitcast(av_v[pl.ds(16 + s * 16, 16)], jnp.float32)
            loc = cur - basev
            m = (loc >= 0) & (loc < chv)
            locc = jnp.where(m, loc, 0)
            jj = jnp.right_shift(locc, 5)
            sw = plsc.load_gather(sw_v, [locc], mask=m)
            vv = plsc.load_gather(v_v, [locc], mask=m)
            ivec = jnp.full((16,), oc + s * 16, jnp.int32) + lanes
            slot = ivec - plsc.load_gather(rs_v, [jj], mask=m)
            slot = jnp.where(m, slot, 0)
            cidx = jnp.right_shift(jnp.bitwise_and(locc, _K - 1), 1)
            plsc.store_scatter(as_v, [slot, jj], a * sw, mask=m)
            plsc.store_scatter(vs_v, [slot, jj], vv * sw, mask=m)
            plsc.store_scatter(cs_v, [slot, jj], cidx, mask=m)
            return ovf_in | jnp.where(m & (slot >= _S1), 1, 0)

        return lax.fori_loop(0, s_hi, body, ovf)

    ovf = lax.fori_loop(0, nch, chunkB, jnp.zeros((16,), jnp.int32))

    flag = jnp.max(ovf)
    rs_v[pl.ds(0, 16)] = jnp.full((16,), flag, jnp.int32)
    o1 = pltpu.async_copy(as_v, as_hbm.at[:, pl.ds(wid * _JT, _JT)], sem)
    o2 = pltpu.async_copy(vs_v, vs_hbm.at[:, pl.ds(wid * _JT, _JT)], sem)
    o3 = pltpu.async_copy(cs_v, cs_hbm.at[:, pl.ds(wid * _JT, _JT)], sem)
    o4 = pltpu.async_copy(rs_v.at[pl.ds(0, 16)], fl_hbm.at[pl.ds(wid * 16, 16)], sem)
    o1.wait()
    o2.wait()
    o3.wait()
    o4.wait()


@functools.cache
def _get_sc_scatter():
    return pl.kernel(
        _sc_scatter_body,
        mesh=plsc.VectorSubcoreMesh(core_axis_name="c", subcore_axis_name="s"),
        compiler_params=pltpu.CompilerParams(needs_layout_passes=False),
        out_type=[
            jax.ShapeDtypeStruct((_K, _J), jnp.float32),
            jax.ShapeDtypeStruct((_K, _J), jnp.float32),
            jax.ShapeDtypeStruct((_K, _J), jnp.int32),
            jax.ShapeDtypeStruct((_NTILES * 16,), jnp.int32),
        ],
        scratch_types=[
            pltpu.VMEM((_WCH + 16,), jnp.int32),
            pltpu.VMEM((_WCH + 16,), jnp.int32),
            pltpu.VMEM((_CH,), jnp.float32),
            pltpu.VMEM((_CH,), jnp.float32),
            pltpu.VMEM((_NTILES + 16,), jnp.int32),
            pltpu.VMEM((_JT,), jnp.int32),
            pltpu.VMEM((_K, _JT), jnp.float32),
            pltpu.VMEM((_K, _JT), jnp.float32),
            pltpu.VMEM((_K, _JT), jnp.int32),
            pltpu.SemaphoreType.DMA,
            pltpu.SemaphoreType.DMA,
        ],
    )


def _slot_terms(xa_full, a_blk, v_blk, c_blk, s_range, iota_c):
    out = None
    for s in s_range:
        ci = c_blk[s]
        b = jnp.where(iota_c == ci[None, :], a_blk[s][None, :], 0.0)
        xa = jnp.dot(xa_full, b, preferred_element_type=jnp.float32)
        t = jnp.maximum(xa - v_blk[s][None, :], 0.0)
        out = t * t if out is None else out + t * t
    return out


def _tc_body(fl_ref, xa_ref, a_ref, v_ref, c_ref, f_ref, o_ref, acc_ref):
    jb = pl.program_id(0)

    xa_full = xa_ref[...]
    a_blk = a_ref[...]
    v_blk = v_ref[...]
    c_blk = c_ref[...]
    iota_c = lax.broadcasted_iota(jnp.int32, (_K, _JB), 0)

    acc_ref[...] = _slot_terms(xa_full, a_blk, v_blk, c_blk,
                               range(_S1), iota_c)

    blockflag = (fl_ref[(4 * jb) * 16] + fl_ref[(4 * jb + 1) * 16]
                 + fl_ref[(4 * jb + 2) * 16] + fl_ref[(4 * jb + 3) * 16])

    @pl.when(blockflag > 0)
    def _overflow():
        acc_ref[...] += _slot_terms(xa_full, a_blk, v_blk, c_blk,
                                    range(_S1, _K), iota_c)

    delta = jnp.exp(-acc_ref[...])
    part = jnp.sum(delta * f_ref[...][None, :], axis=1)

    @pl.when(jb == 0)
    def _zero_out():
        o_ref[...] = jnp.zeros_like(o_ref)

    o_ref[...] += part


_tc_compute = pl.pallas_call(
    _tc_body,
    grid=(_J // _JB,),
    in_specs=[
        pl.BlockSpec(memory_space=pltpu.SMEM),
        pl.BlockSpec((_M, _K), lambda jb: (0, 0)),
        pl.BlockSpec((_K, _JB), lambda jb: (0, jb)),
        pl.BlockSpec((_K, _JB), lambda jb: (0, jb)),
        pl.BlockSpec((_K, _JB), lambda jb: (0, jb)),
        pl.BlockSpec((_JB,), lambda jb: (jb,)),
    ],
    out_specs=pl.BlockSpec((_M,), lambda jb: (0,)),
    out_shape=jax.ShapeDtypeStruct((_M,), jnp.float32),
    scratch_shapes=[pltpu.VMEM((_M, _JB), jnp.float32)],
)


def kernel(X, a_vals, rows, cols, V, F_vec, W):
    rows = rows.astype(jnp.int32)
    cols = cols.astype(jnp.int32)
    f = rows * _K + cols
    f_e = jnp.pad(f, (16, _NNZ_PAD - _NNZ), constant_values=-1)
    av_e = jnp.pad(a_vals, (16, _NNZ_PAD - _NNZ))
    fav = jnp.concatenate([f_e, lax.bitcast_convert_type(av_e, jnp.int32)])
    bnds = jnp.arange(_NTILES + 16, dtype=jnp.int32) * _CH
    off = jnp.sum((f[None, :] < bnds[:, None]).astype(jnp.int32), axis=1)
    swv = jnp.concatenate([jnp.sqrt(W).reshape(-1), V.reshape(-1)])
    a_s, v_s, c_s, flags = _get_sc_scatter()(fav, swv, off)
    x_aug = jnp.concatenate(
        [X, jnp.ones((_M, 1), jnp.float32),
         jnp.zeros((_M, _K - X.shape[1] - 1), jnp.float32)], axis=1)
    return _tc_compute(flags, x_aug, a_s, v_s, c_s, F_vec)


# speedup vs baseline: 1.0356x; 1.0356x over previous
import functools

import jax
import jax.numpy as jnp
from jax import lax
from jax.experimental import pallas as pl
from jax.experimental.pallas import tpu as pltpu
from jax.experimental.pallas import tpu_sc as plsc

_M = 256
_J = 4096
_K = 32
_NNZ = 13107
_NNZ_PAD = 13120
_EXT = _NNZ_PAD + 16
_NTILES = 32
_JT = _J // _NTILES
_CH = _JT * _K
_S1 = 12

_JB = 1024

_WCH = 2048


def _sc_scatter_body(fav_hbm, swv_hbm, off_hbm,
                     as_hbm, vs_hbm, cs_hbm, fl_hbm,
                     f_v, av_v, sw_v, v_v, off_v, rs_v, as_v, vs_v, cs_v,
                     sem, sem2):
    nc = 2
    wid = lax.axis_index("s") * nc + lax.axis_index("c")
    base = wid * _CH
    c_off = pltpu.async_copy(off_hbm, off_v, sem2)
    c_sw = pltpu.async_copy(swv_hbm.at[pl.ds(base, _CH)], sw_v, sem)
    c_v = pltpu.async_copy(swv_hbm.at[pl.ds(_K * _J + base, _CH)], v_v, sem)

    zf = jnp.zeros((16,), jnp.float32)
    zi = jnp.zeros((16,), jnp.int32)

    def zero_body(i, c):
        r = i >> 3
        col = (i & 7) * 16
        as_v[r, pl.ds(col, 16)] = zf
        vs_v[r, pl.ds(col, 16)] = zf
        cs_v[r, pl.ds(col, 16)] = zi
        return c

    lax.fori_loop(0, _K * _JT // 16, zero_body, 0)
    c_off.wait()

    offs = off_v[pl.ds(wid, 16)]
    lo = offs[0]
    hi = offs[1]
    lo_fl = (lo // 16) * 16
    nch = (hi - lo_fl + _WCH - 1) // _WCH
    basev = jnp.full((16,), base, jnp.int32)
    chv = jnp.full((16,), _CH, jnp.int32)
    lanes = lax.iota(jnp.int32, 16)

    def chunkA(c, carry):
        oc = jnp.minimum(lo_fl + c * _WCH, _EXT - (_WCH + 16))
        pltpu.sync_copy(fav_hbm.at[pl.ds(oc, _WCH + 16)], f_v)
        s_hi = jnp.minimum((hi - oc + 15) // 16, _WCH // 16)

        def body(s, cc):
            cur = f_v[pl.ds(16 + s * 16, 16)]
            prv = f_v[pl.ds(15 + s * 16, 16)]
            loc = cur - basev
            m = (loc >= 0) & (loc < chv)
            jj = jnp.right_shift(jnp.where(m, loc, 0), 5)
            first = m & (jnp.right_shift(cur, 5) != jnp.right_shift(prv, 5))
            ivec = jnp.full((16,), oc + s * 16, jnp.int32) + lanes
            plsc.store_scatter(rs_v, [jj], ivec, mask=first)
            return cc

        lax.fori_loop(0, s_hi, body, 0)
        return carry

    lax.fori_loop(0, nch, chunkA, 0)
    c_sw.wait()
    c_v.wait()

    def chunkB(c, ovf):
        oc = jnp.minimum(lo_fl + c * _WCH, _EXT - (_WCH + 16))

        @pl.when(nch > 1)
        def _refetch():
            pltpu.sync_copy(fav_hbm.at[pl.ds(oc, _WCH + 16)], f_v)

        pltpu.sync_copy(fav_hbm.at[pl.ds(_EXT + oc, _WCH + 16)], av_v)
        s_hi = jnp.minimum((hi - oc + 15) // 16, _WCH // 16)

        def body(s, ovf_in):
            cur = f_v[pl.ds(16 + s * 16, 16)]
            a = plsc.bitcast(av_v[pl.ds(16 + s * 16, 16)], jnp.float32)
            loc = cur - basev
            m = (loc >= 0) & (loc < chv)
            locc = jnp.where(m, loc, 0)
            jj = jnp.right_shift(locc, 5)
            sw = plsc.load_gather(sw_v, [locc], mask=m)
            vv = plsc.load_gather(v_v, [locc], mask=m)
            ivec = jnp.full((16,), oc + s * 16, jnp.int32) + lanes
            slot = ivec - plsc.load_gather(rs_v, [jj], mask=m)
            slot = jnp.where(m, slot, 0)
            cidx = jnp.right_shift(jnp.bitwise_and(locc, _K - 1), 1)
            plsc.store_scatter(as_v, [slot, jj], a * sw, mask=m)
            plsc.store_scatter(vs_v, [slot, jj], vv * sw, mask=m)
            plsc.store_scatter(cs_v, [slot, jj], cidx, mask=m)
            return ovf_in | jnp.where(m & (slot >= _S1), 1, 0)

        return lax.fori_loop(0, s_hi, body, ovf)

    ovf = lax.fori_loop(0, nch, chunkB, jnp.zeros((16,), jnp.int32))

    flag = jnp.max(ovf)
    rs_v[pl.ds(0, 16)] = jnp.full((16,), flag, jnp.int32)
    o1 = pltpu.async_copy(as_v, as_hbm.at[:, pl.ds(wid * _JT, _JT)], sem)
    o2 = pltpu.async_copy(vs_v, vs_hbm.at[:, pl.ds(wid * _JT, _JT)], sem)
    o3 = pltpu.async_copy(cs_v, cs_hbm.at[:, pl.ds(wid * _JT, _JT)], sem)
    o4 = pltpu.async_copy(rs_v.at[pl.ds(0, 16)], fl_hbm.at[pl.ds(wid * 16, 16)], sem)
    o1.wait()
    o2.wait()
    o3.wait()
    o4.wait()


@functools.cache
def _get_sc_scatter():
    return pl.kernel(
        _sc_scatter_body,
        mesh=plsc.VectorSubcoreMesh(core_axis_name="c", subcore_axis_name="s"),
        compiler_params=pltpu.CompilerParams(needs_layout_passes=False),
        out_type=[
            jax.ShapeDtypeStruct((_K, _J), jnp.float32),
            jax.ShapeDtypeStruct((_K, _J), jnp.float32),
            jax.ShapeDtypeStruct((_K, _J), jnp.int32),
            jax.ShapeDtypeStruct((_NTILES * 16,), jnp.int32),
        ],
        scratch_types=[
            pltpu.VMEM((_WCH + 16,), jnp.int32),
            pltpu.VMEM((_WCH + 16,), jnp.int32),
            pltpu.VMEM((_CH,), jnp.float32),
            pltpu.VMEM((_CH,), jnp.float32),
            pltpu.VMEM((_NTILES + 16,), jnp.int32),
            pltpu.VMEM((_JT,), jnp.int32),
            pltpu.VMEM((_K, _JT), jnp.float32),
            pltpu.VMEM((_K, _JT), jnp.float32),
            pltpu.VMEM((_K, _JT), jnp.int32),
            pltpu.SemaphoreType.DMA,
            pltpu.SemaphoreType.DMA,
        ],
    )


def _slot_terms(xa_full, a_blk, v_blk, c_blk, s_range, iota_c):
    out = None
    for s in s_range:
        ci = c_blk[s]
        b = jnp.where(iota_c == ci[None, :], a_blk[s][None, :], 0.0)
        xa = jnp.dot(xa_full, b, preferred_element_type=jnp.float32)
        t = jnp.maximum(xa - v_blk[s][None, :], 0.0)
        out = t * t if out is None else out + t * t
    return out


def _tc_body(fl_ref, xa_ref, a_ref, v_ref, c_ref, f_ref, o_ref, acc_ref):
    jb = pl.program_id(0)

    xa_full = xa_ref[...]
    a_blk = a_ref[...]
    v_blk = v_ref[...]
    c_blk = c_ref[...]
    iota_c = lax.broadcasted_iota(jnp.int32, (_K, _JB), 0)

    acc_ref[...] = _slot_terms(xa_full, a_blk, v_blk, c_blk,
                               range(_S1), iota_c)

    blockflag = (fl_ref[(4 * jb) * 16] + fl_ref[(4 * jb + 1) * 16]
                 + fl_ref[(4 * jb + 2) * 16] + fl_ref[(4 * jb + 3) * 16])

    @pl.when(blockflag > 0)
    def _overflow():
        acc_ref[...] += _slot_terms(xa_full, a_blk, v_blk, c_blk,
                                    range(_S1, _K), iota_c)

    delta = jnp.exp(-acc_ref[...])
    part = jnp.sum(delta * f_ref[...][None, :], axis=1)

    @pl.when(jb == 0)
    def _zero_out():
        o_ref[...] = jnp.zeros_like(o_ref)

    o_ref[...] += part


_tc_compute = pl.pallas_call(
    _tc_body,
    grid=(_J // _JB,),
    in_specs=[
        pl.BlockSpec(memory_space=pltpu.SMEM),
        pl.BlockSpec((_M, _K), lambda jb: (0, 0)),
        pl.BlockSpec((_K, _JB), lambda jb: (0, jb)),
        pl.BlockSpec((_K, _JB), lambda jb: (0, jb)),
        pl.BlockSpec((_K, _JB), lambda jb: (0, jb)),
        pl.BlockSpec((_JB,), lambda jb: (jb,)),
    ],
    out_specs=pl.BlockSpec((_M,), lambda jb: (0,)),
    out_shape=jax.ShapeDtypeStruct((_M,), jnp.float32),
    scratch_shapes=[pltpu.VMEM((_M, _JB), jnp.float32)],
)


def kernel(X, a_vals, rows, cols, V, F_vec, W):
    rows = rows.astype(jnp.int32)
    cols = cols.astype(jnp.int32)
    f = rows * _K + cols
    f_e = jnp.pad(f, (16, _NNZ_PAD - _NNZ), constant_values=-1)
    av_e = jnp.pad(a_vals, (16, _NNZ_PAD - _NNZ))
    fav = jnp.concatenate([f_e, lax.bitcast_convert_type(av_e, jnp.int32)])
    bnds = jnp.arange(_NTILES + 16, dtype=jnp.int32) * _CH
    off = jnp.sum((f[None, :] < bnds[:, None]).astype(jnp.int32), axis=1)
    swv = jnp.concatenate([jnp.sqrt(W).reshape(-1), V.reshape(-1)])
    a_s, v_s, c_s, flags = _get_sc_scatter()(fav, swv, off)
    x_aug = jnp.concatenate(
        [X, jnp.ones((_M, 1), jnp.float32),
         jnp.zeros((_M, _K - X.shape[1] - 1), jnp.float32)], axis=1)
    return _tc_compute(flags, x_aug, a_s, v_s, c_s, F_vec)
